# baseline (speedup 1.0000x reference)
import functools

import jax
import jax.numpy as jnp
import numpy as np
from jax import lax
from jax.experimental import pallas as pl
from jax.experimental.pallas import tpu as pltpu
from jax.experimental.pallas import tpu_sc as plsc

N = 4096
B = 1024
L = 50
LP = 64
LL = L * L
LL_PAD = 2560
CHUNKS = LL_PAD // 16
DMA_ROWS = LL_PAD // 128
TAIL = LL - (DMA_ROWS - 1) * 128
VCHUNKS = (LL + 15) // 16

_info = plsc.get_sparse_core_info()
NC, NS = _info.num_cores, _info.num_subcores
NW = NC * NS
B_PER_W = B // NW

_k = np.arange(LL_PAD)
_i_sel = np.where(_k < LL, _k // L, 0).astype(np.int32)
_j_sel = np.where(_k < LL, _k % L, 0).astype(np.int32)


def _sc_kernel(table_hbm, idx_hbm, isel_hbm, jsel_hbm, out_hbm,
               ind_v, isel_v, jsel_v, rbase_v, cbase_v,
               fa0, fa1, fb0, fb1, vals_v, dummy_v, sem_a, sem_b):
    wid = lax.axis_index("s") * NC + lax.axis_index("c")
    base = wid * B_PER_W
    pltpu.sync_copy(isel_hbm, isel_v)
    pltpu.sync_copy(jsel_hbm, jsel_v)
    pltpu.sync_copy(idx_hbm.at[pl.ds(base, B_PER_W)], ind_v)

    def compute_flat(t, flat_ref):
        tv = jnp.full((16,), 0, jnp.int32) + t
        for c in range(LP // 16):
            lanes = jnp.minimum(lax.iota(jnp.int32, 16) + c * 16, L - 1)
            iv = plsc.load_gather(ind_v, [tv, lanes])
            rbase_v[pl.ds(c * 16, 16)] = (
                lax.shift_left(lax.shift_right_logical(iv, 3), 15)
                + lax.shift_left(lax.bitwise_and(iv, 7), 7))
            cbase_v[pl.ds(c * 16, 16)] = (
                lax.shift_left(lax.shift_right_logical(iv, 7), 10)
                + lax.bitwise_and(iv, 127))
        for c in range(VCHUNKS):
            rs = isel_v[pl.ds(c * 16, 16)]
            cs = jsel_v[pl.ds(c * 16, 16)]
            rv = plsc.load_gather(rbase_v, [rs])
            cv = plsc.load_gather(cbase_v, [cs])
            flat_ref[c // 8, pl.ds((c % 8) * 16, 16)] = rv + cv

    def fire(flat_ref, t, sem):
        for r in range(DMA_ROWS - 1):
            pltpu.async_copy(table_hbm.at[flat_ref.at[r]],
                             vals_v.at[t, pl.ds(r * 128, 128)], sem)
        pltpu.async_copy(
            table_hbm.at[flat_ref.at[DMA_ROWS - 1, pl.ds(0, TAIL)]],
            vals_v.at[t, pl.ds((DMA_ROWS - 1) * 128, TAIL)], sem)

    def compute_and_fire(t, f0, f1, sem):
        compute_flat(t, f0)
        fire(f0, t, sem)
        compute_flat(t + 1, f1)
        fire(f1, t + 1, sem)

    def drain(sem):
        pltpu.make_async_copy(
            table_hbm.at[pl.ds(0, 2 * LL)], dummy_v, sem).wait()

    compute_and_fire(0, fa0, fa1, sem_a)

    def body(v, carry):
        compute_and_fire(4 * v + 2, fb0, fb1, sem_b)
        drain(sem_a)
        compute_and_fire(4 * v + 4, fa0, fa1, sem_a)
        drain(sem_b)
        return carry

    lax.fori_loop(0, 7, body, 0)
    compute_and_fire(B_PER_W - 2, fb0, fb1, sem_b)
    drain(sem_a)
    drain(sem_b)
    pltpu.sync_copy(vals_v, out_hbm.at[pl.ds(base, B_PER_W)])


@jax.jit
def _run(table_flat, idx32, isel, jsel):
    mesh = plsc.VectorSubcoreMesh(core_axis_name="c", subcore_axis_name="s")
    k = functools.partial(
        pl.kernel,
        mesh=mesh,
        compiler_params=pltpu.CompilerParams(needs_layout_passes=False),
        out_type=jax.ShapeDtypeStruct((B, LL_PAD), jnp.float32),
        scratch_types=[
            pltpu.VMEM((B_PER_W, L), jnp.int32),
            pltpu.VMEM((LL_PAD,), jnp.int32),
            pltpu.VMEM((LL_PAD,), jnp.int32),
            pltpu.VMEM((LP,), jnp.int32),
            pltpu.VMEM((LP,), jnp.int32),
            pltpu.VMEM((DMA_ROWS, 128), jnp.int32),
            pltpu.VMEM((DMA_ROWS, 128), jnp.int32),
            pltpu.VMEM((DMA_ROWS, 128), jnp.int32),
            pltpu.VMEM((DMA_ROWS, 128), jnp.int32),
            pltpu.VMEM((B_PER_W, LL_PAD), jnp.float32),
            pltpu.VMEM((2 * LL,), jnp.float32),
            pltpu.SemaphoreType.DMA,
            pltpu.SemaphoreType.DMA,
        ],
    )(_sc_kernel)
    return k(table_flat, idx32, isel, jsel)


def kernel(innerVars, indices):
    table_flat = (
        innerVars.reshape(N // 8, 8, N // 128, 128)
        .transpose(0, 2, 1, 3)
        .reshape(-1)
    )
    idx32 = indices.astype(jnp.int32)
    out = _run(table_flat, idx32, jnp.asarray(_i_sel), jnp.asarray(_j_sel))
    one = jnp.sign(jnp.abs(innerVars[0, 0]) + jnp.float32(1.0))
    return (out[:, :LL] * one).reshape(B, L, L)


# speedup vs baseline: 1.0548x; 1.0548x over previous
import functools

import jax
import jax.numpy as jnp
import numpy as np
from jax import lax
from jax.experimental import pallas as pl
from jax.experimental.pallas import tpu as pltpu
from jax.experimental.pallas import tpu_sc as plsc

N = 4096
B = 1024
L = 50
LP = 64
LL = L * L
LL_PAD = 2560
CHUNKS = LL_PAD // 16
DMA_ROWS = LL_PAD // 128
TAIL = LL - (DMA_ROWS - 1) * 128
VCHUNKS = (LL + 15) // 16

_info = plsc.get_sparse_core_info()
NC, NS = _info.num_cores, _info.num_subcores
NW = NC * NS
B_PER_W = B // NW

_k = np.arange(LL_PAD)
_i_sel = np.where(_k < LL, _k // L, 0).astype(np.int32)
_j_sel = np.where(_k < LL, _k % L, 0).astype(np.int32)
_sel = np.concatenate([_i_sel, _j_sel])


def _sc_kernel(table_hbm, idx_hbm, sel_hbm, out_hbm,
               ind_v, sel_v, rbase_v, cbase_v,
               fa0, fa1, fb0, fb1, vals_v, dummy_v, sem_a, sem_b):
    wid = lax.axis_index("s") * NC + lax.axis_index("c")
    base = wid * B_PER_W
    cp_i = pltpu.async_copy(idx_hbm.at[pl.ds(base, B_PER_W)], ind_v, sem_a)
    cp_s = pltpu.async_copy(sel_hbm, sel_v, sem_a)
    cp_i.wait()
    cp_s.wait()

    def compute_flat(t, flat_ref):
        tv = jnp.full((16,), 0, jnp.int32) + t
        for c in range(LP // 16):
            lanes = jnp.minimum(lax.iota(jnp.int32, 16) + c * 16, L - 1)
            iv = plsc.load_gather(ind_v, [tv, lanes])
            rbase_v[pl.ds(c * 16, 16)] = (
                lax.shift_left(lax.shift_right_logical(iv, 3), 15)
                + lax.shift_left(lax.bitwise_and(iv, 7), 7))
            cbase_v[pl.ds(c * 16, 16)] = (
                lax.shift_left(lax.shift_right_logical(iv, 7), 10)
                + lax.bitwise_and(iv, 127))
        for c in range(VCHUNKS):
            rs = sel_v[pl.ds(c * 16, 16)]
            cs = sel_v[pl.ds(LL_PAD + c * 16, 16)]
            rv = plsc.load_gather(rbase_v, [rs])
            cv = plsc.load_gather(cbase_v, [cs])
            flat_ref[c // 8, pl.ds((c % 8) * 16, 16)] = rv + cv

    def fire(flat_ref, t, sem):
        for r in range(DMA_ROWS - 1):
            pltpu.async_copy(table_hbm.at[flat_ref.at[r]],
                             vals_v.at[t, pl.ds(r * 128, 128)], sem)
        pltpu.async_copy(
            table_hbm.at[flat_ref.at[DMA_ROWS - 1, pl.ds(0, TAIL)]],
            vals_v.at[t, pl.ds((DMA_ROWS - 1) * 128, TAIL)], sem)

    def compute_and_fire(t, f0, f1, sem):
        compute_flat(t, f0)
        fire(f0, t, sem)
        compute_flat(t + 1, f1)
        fire(f1, t + 1, sem)

    def drain(sem):
        pltpu.make_async_copy(
            table_hbm.at[pl.ds(0, 2 * LL)], dummy_v, sem).wait()

    compute_and_fire(0, fa0, fa1, sem_a)

    def body(v, carry):
        compute_and_fire(4 * v + 2, fb0, fb1, sem_b)
        drain(sem_a)
        compute_and_fire(4 * v + 4, fa0, fa1, sem_a)
        drain(sem_b)
        return carry

    lax.fori_loop(0, 7, body, 0)
    compute_and_fire(B_PER_W - 2, fb0, fb1, sem_b)
    drain(sem_a)
    drain(sem_b)
    pltpu.sync_copy(vals_v, out_hbm.at[pl.ds(base, B_PER_W)])


@jax.jit
def _run(table_flat, idx32, sel):
    mesh = plsc.VectorSubcoreMesh(core_axis_name="c", subcore_axis_name="s")
    k = functools.partial(
        pl.kernel,
        mesh=mesh,
        compiler_params=pltpu.CompilerParams(needs_layout_passes=False),
        out_type=jax.ShapeDtypeStruct((B, LL_PAD), jnp.float32),
        scratch_types=[
            pltpu.VMEM((B_PER_W, L), jnp.int32),
            pltpu.VMEM((2 * LL_PAD,), jnp.int32),
            pltpu.VMEM((LP,), jnp.int32),
            pltpu.VMEM((LP,), jnp.int32),
            pltpu.VMEM((DMA_ROWS, 128), jnp.int32),
            pltpu.VMEM((DMA_ROWS, 128), jnp.int32),
            pltpu.VMEM((DMA_ROWS, 128), jnp.int32),
            pltpu.VMEM((DMA_ROWS, 128), jnp.int32),
            pltpu.VMEM((B_PER_W, LL_PAD), jnp.float32),
            pltpu.VMEM((2 * LL,), jnp.float32),
            pltpu.SemaphoreType.DMA,
            pltpu.SemaphoreType.DMA,
        ],
    )(_sc_kernel)
    return k(table_flat, idx32, sel)


def kernel(innerVars, indices):
    table_flat = (
        innerVars.reshape(N // 8, 8, N // 128, 128)
        .transpose(0, 2, 1, 3)
        .reshape(-1)
    )
    idx32 = indices.astype(jnp.int32)
    out = _run(table_flat, idx32, jnp.asarray(_sel))
    return out[:, :LL].reshape(B, L, L)
